# baseline (speedup 1.0000x reference)
import functools

import jax
import jax.numpy as jnp
from jax import lax
from jax.experimental import pallas as pl
from jax.experimental.pallas import tpu as pltpu
from jax.experimental.pallas import tpu_sc as plsc

N = 10000
E = 320000
H = 4
C = 128
D_IN = 128
D_E = 16
D_OUT = 128
HC = H * C

_NC = 2
_NS = 16
_NW = _NC * _NS
CH = 128
NCHUNK = E // CH
_CPW = NCHUNK // _NW
_CREM = NCHUNK - _CPW * _NW
_RPT = 624
_RREM = N - _RPT * _NS
CC = 16
NCHUNK_C = E // CC
_CPT = NCHUNK_C // _NW

_f32 = jnp.float32
_i32 = jnp.int32


_BN = 1000


def _prep_node_body(x_ref, wsrc_ref, attsrc_ref, attdst_ref, wlin_ref,
                    ys_ref, meta_ref):
    xs = jnp.dot(x_ref[...], wsrc_ref[...], preferred_element_type=_f32)
    acols = []
    dcols = []
    for h in range(H):
        blk = xs[:, h * C:(h + 1) * C]
        acols.append((blk * attsrc_ref[h:h + 1, :]).sum(axis=1, keepdims=True))
        dcols.append((blk * attdst_ref[h:h + 1, :]).sum(axis=1, keepdims=True))
        ys_ref[:, h * C:(h + 1) * C] = jnp.dot(
            blk, wlin_ref[h * C:(h + 1) * C, :], preferred_element_type=_f32)
    meta_ref[...] = jnp.concatenate(acols + dcols, axis=1)


def _prep_node(x, W_src, att_src, att_dst, W_lin):
    return pl.pallas_call(
        _prep_node_body,
        grid=(N // _BN,),
        in_specs=[
            pl.BlockSpec((_BN, D_IN), lambda i: (i, 0)),
            pl.BlockSpec((D_IN, HC), lambda i: (0, 0)),
            pl.BlockSpec((H, C), lambda i: (0, 0)),
            pl.BlockSpec((H, C), lambda i: (0, 0)),
            pl.BlockSpec((HC, D_OUT), lambda i: (0, 0)),
        ],
        out_specs=[
            pl.BlockSpec((_BN, HC), lambda i: (i, 0)),
            pl.BlockSpec((_BN, 2 * H), lambda i: (i, 0)),
        ],
        out_shape=[
            jax.ShapeDtypeStruct((N, HC), _f32),
            jax.ShapeDtypeStruct((N, 2 * H), _f32),
        ],
    )(x, W_src, att_src, att_dst, W_lin)


_BE = 12800


def _prep_edge_body(ea_ref, wedge_ref, attedge_ref, out_ref):
    cols = []
    for h in range(H):
        cols.append((wedge_ref[:, h * C:(h + 1) * C]
                     * attedge_ref[h:h + 1, :]).sum(axis=1, keepdims=True))
    ae_fold = jnp.concatenate(cols, axis=1)
    out_ref[...] = lax.dot_general(
        ae_fold, ea_ref[...], (((0,), (1,)), ((), ())),
        preferred_element_type=_f32)


def _prep_edge(edge_attr, W_edge, att_edge):
    return pl.pallas_call(
        _prep_edge_body,
        grid=(E // _BE,),
        in_specs=[
            pl.BlockSpec((_BE, D_E), lambda i: (i, 0)),
            pl.BlockSpec((D_E, HC), lambda i: (0, 0)),
            pl.BlockSpec((H, C), lambda i: (0, 0)),
        ],
        out_specs=pl.BlockSpec((H, _BE), lambda i: (0, i)),
        out_shape=jax.ShapeDtypeStruct((H, E), _f32),
    )(edge_attr, W_edge, att_edge)


def _mid_body(acc_ref, meta_ref, invd_ref, aself_ref):
    acc = acc_ref[:N, :] + acc_ref[N:, :]
    denom_e = acc[:, 0:H]
    aesum = acc[:, H:2 * H]
    deg = acc[:, 2 * H:2 * H + 1]
    ae_self = aesum / jnp.maximum(deg, 1.0)
    a = meta_ref[:, 0:H] + meta_ref[:, H:2 * H] + ae_self
    a = jnp.where(a >= 0.0, a, 0.2 * a)
    p_self = jnp.exp(a)
    invd = 1.0 / (denom_e + p_self + 1e-16)
    invd_ref[...] = jnp.concatenate(
        [invd, jnp.zeros((N, 16 - H), _f32)], axis=1)
    aself_ref[...] = p_self * invd


def _mid(accA, meta):
    return pl.pallas_call(
        _mid_body,
        out_shape=[
            jax.ShapeDtypeStruct((N, 16), _f32),
            jax.ShapeDtypeStruct((N, H), _f32),
        ],
    )(accA, meta)


def _post_body(x_ref, a0_ref, a1_ref, aself_ref, ys_ref, wlin_ref,
               bg_ref, bl_ref, g_ref, b_ref, out_ref):
    c0 = jnp.dot(bg_ref[...], wlin_ref[...], preferred_element_type=_f32) \
        + bl_ref[...]
    st = a0_ref[...] + a1_ref[...]
    for h in range(H):
        st = st + aself_ref[:, h:h + 1] * ys_ref[:, h * C:(h + 1) * C]
    hh = x_ref[...] + st + c0
    mu = jnp.mean(hh, axis=1, keepdims=True)
    var = jnp.mean((hh - mu) ** 2, axis=1, keepdims=True)
    out_ref[...] = (hh - mu) * lax.rsqrt(var + 1e-5) * g_ref[...] + b_ref[...]


def _post(x, accC, aself, ys, W_lin, bias_gat, b_lin, gamma, beta):
    nb = N // _BN
    return pl.pallas_call(
        _post_body,
        grid=(nb,),
        in_specs=[
            pl.BlockSpec((_BN, D_IN), lambda i: (i, 0)),
            pl.BlockSpec((_BN, D_OUT), lambda i: (i, 0)),
            pl.BlockSpec((_BN, D_OUT), lambda i, _nb=nb: (i + _nb, 0)),
            pl.BlockSpec((_BN, H), lambda i: (i, 0)),
            pl.BlockSpec((_BN, HC), lambda i: (i, 0)),
            pl.BlockSpec((HC, D_OUT), lambda i: (0, 0)),
            pl.BlockSpec((1, HC), lambda i: (0, 0)),
            pl.BlockSpec((1, D_OUT), lambda i: (0, 0)),
            pl.BlockSpec((1, D_OUT), lambda i: (0, 0)),
            pl.BlockSpec((1, D_OUT), lambda i: (0, 0)),
        ],
        out_specs=pl.BlockSpec((_BN, D_OUT), lambda i: (i, 0)),
        out_shape=jax.ShapeDtypeStruct((N, D_OUT), _f32),
    )(x, accC, accC, aself, ys, W_lin, bias_gat, b_lin, gamma, beta)


_mesh = plsc.VectorSubcoreMesh(core_axis_name="c", subcore_axis_name="s")


def _tile_chunk_range(cid, sid, nchunk=NCHUNK):
    wid = sid * _NC + cid
    cpw = nchunk // _NW
    crem = nchunk - cpw * _NW
    start = wid * cpw + jnp.minimum(wid, crem)
    count = jnp.where(wid < crem, cpw + 1, cpw)
    return start, count


_TAKE_DN = lax.GatherDimensionNumbers(
    offset_dims=(), collapsed_slice_dims=(0,), start_index_map=(0,))


def _lane_take(x, idx):
    return lax.gather(
        x, idx[:, None], _TAKE_DN, slice_sizes=(1,),
        mode=lax.GatherScatterMode.PROMISE_IN_BOUNDS)


def _copy_out_rows(sid, sh_ref, hbm_ref, row0):
    pltpu.sync_copy(
        sh_ref.at[pl.ds(sid * _RPT, _RPT)],
        hbm_ref.at[pl.ds(row0 + sid * _RPT, _RPT)])

    @pl.when(sid == 0)
    def _():
        pltpu.sync_copy(
            sh_ref.at[pl.ds(_NS * _RPT, _RREM)],
            hbm_ref.at[pl.ds(row0 + _NS * _RPT, _RREM)])


CA = 80
NCHUNK_A = E // CA
_APT = NCHUNK_A // _NW


@functools.partial(
    pl.kernel,
    mesh=_mesh,
    compiler_params=pltpu.CompilerParams(
        needs_layout_passes=False, use_tc_tiling_on_sc=False),
    out_type=[
        jax.ShapeDtypeStruct((E * H,), _f32),
        jax.ShapeDtypeStruct((_NC * N, 16), _f32),
    ],
    scratch_types=(
        [pltpu.VMEM((N * 2 * H,), _f32)]
        + [pltpu.VMEM((CA,), _i32)] * 2
        + [pltpu.VMEM((CA,), _i32)] * 2
        + [pltpu.VMEM((H, CA), _f32)] * 2
        + [
            pltpu.VMEM((CA * H,), _f32),
            pltpu.VMEM((CA, 16), _f32),
            pltpu.VMEM_SHARED((N, 16), _f32),
            pltpu.SemaphoreType.DMA,
            pltpu.SemaphoreType.DMA,
        ]
    ),
)
def _pass_a(src_hbm, dst_hbm, aeT_hbm, meta_hbm, z16_hbm,
            pT_hbm, acc_hbm,
            meta_v, srcv0, srcv1, dstv0, dstv1, aev0, aev1,
            pst, upd, acc_sh, sem0, sem1):
    cid = lax.axis_index("c")
    sid = lax.axis_index("s")
    wid = sid * _NC + cid
    base = wid * _APT
    srcv = (srcv0, srcv1)
    dstv = (dstv0, dstv1)
    aev = (aev0, aev1)
    sem = (sem0, sem1)
    pltpu.sync_copy(meta_hbm, meta_v)
    zero16 = jnp.zeros((16,), _f32)
    for r in range(CA):
        upd[r, :] = zero16

    @pl.when(sid == 0)
    def _():
        pltpu.sync_copy(z16_hbm, acc_sh)

    plsc.subcore_barrier()
    iota16 = lax.iota(_i32, 16)
    ones16 = jnp.ones((16,), _f32)

    def load_in(ci, b):
        off = ci * CA
        pltpu.async_copy(src_hbm.at[pl.ds(off, CA)], srcv[b], sem[b])
        pltpu.async_copy(dst_hbm.at[pl.ds(off, CA)], dstv[b], sem[b])
        pltpu.async_copy(aeT_hbm.at[:, pl.ds(off, CA)], aev[b], sem[b])

    def wait_in(ci, b):
        off = ci * CA
        pltpu.make_async_copy(src_hbm.at[pl.ds(off, CA)], srcv[b],
                              sem[b]).wait()
        pltpu.make_async_copy(dst_hbm.at[pl.ds(off, CA)], dstv[b],
                              sem[b]).wait()
        pltpu.make_async_copy(aeT_hbm.at[:, pl.ds(off, CA)], aev[b],
                              sem[b]).wait()

    def compute(ci, b):
        off = ci * CA
        for g in range(CA // 16):
            s16 = srcv[b][pl.ds(g * 16, 16)]
            d16 = dstv[b][pl.ds(g * 16, 16)]
            rowi = iota16 + (g * 16)
            for h in range(H):
                a_s = plsc.load_gather(meta_v, [s16 * (2 * H) + h])
                a_d = plsc.load_gather(meta_v, [d16 * (2 * H) + (H + h)])
                ae = aev[b][h, pl.ds(g * 16, 16)]
                a = a_s + a_d + ae
                a = jnp.where(a >= 0.0, a, 0.2 * a)
                p = jnp.exp(a)
                plsc.store_scatter(pst, [rowi * H + h], p)
                plsc.store_scatter(
                    upd, [rowi, jnp.full((16,), h, _i32)], p)
                plsc.store_scatter(
                    upd, [rowi, jnp.full((16,), H + h, _i32)], ae)
            plsc.store_scatter(
                upd, [rowi, jnp.full((16,), 2 * H, _i32)], ones16)
        pltpu.sync_copy(pst, pT_hbm.at[pl.ds(off * H, CA * H)])
        pltpu.sync_copy(upd, acc_sh.at[dstv[b]], add=True)

    load_in(base, 0)

    def chunk2(i, carry):
        for b in range(2):
            ci = base + i * 2 + b
            load_in(ci + 1, 1 - b)
            wait_in(ci, b)
            compute(ci, b)
        return carry

    lax.fori_loop(0, (_APT - 1) // 2, chunk2, None)
    wait_in(base + _APT - 1, 0)
    compute(base + _APT - 1, 0)
    plsc.subcore_barrier()
    _copy_out_rows(sid, acc_sh, acc_hbm, cid * N)


@functools.partial(
    pl.kernel,
    mesh=_mesh,
    compiler_params=pltpu.CompilerParams(
        needs_layout_passes=False, use_tc_tiling_on_sc=False),
    out_type=jax.ShapeDtypeStruct((_NC * N, D_OUT), _f32),
    scratch_types=(
        [pltpu.VMEM((CC,), _i32)] * 2
        + [pltpu.VMEM((CC,), _i32)] * 2
        + [pltpu.VMEM((CC * H,), _f32)] * 2
        + [pltpu.VMEM((CC, HC), _f32)] * 2
        + [pltpu.VMEM((CC, D_OUT), _f32)] * 2
        + [pltpu.VMEM((CC,), _i32)] * 2
        + [
            pltpu.VMEM((CC, 16), _f32),
            pltpu.VMEM_SHARED((N, D_OUT), _f32),
            pltpu.SemaphoreType.DMA,
            pltpu.SemaphoreType.DMA,
            pltpu.SemaphoreType.DMA,
            pltpu.SemaphoreType.DMA,
        ]
    ),
)
def _pass_c(src_hbm, dst_hbm, pT_hbm, invd_hbm, ys_hbm, z128_hbm,
            out_hbm,
            srcv0, srcv1, dstv0, dstv1, pv0, pv1, rows0, rows1,
            msg0, msg1, sdst0, sdst1, ivd, acc_sh, sem0, sem1, ssem0, ssem1):
    cid = lax.axis_index("c")
    sid = lax.axis_index("s")
    wid = sid * _NC + cid
    base = wid * _CPT
    srcv = (srcv0, srcv1)
    dstv = (dstv0, dstv1)
    pv = (pv0, pv1)
    rows = (rows0, rows1)
    msg = (msg0, msg1)
    sdst = (sdst0, sdst1)
    sem = (sem0, sem1)
    ssem = (ssem0, ssem1)
    iota16 = lax.iota(_i32, 16)
    grow = iota16 // H
    gcol = iota16 - grow * H

    @pl.when(sid == 0)
    def _():
        pltpu.sync_copy(z128_hbm, acc_sh)

    plsc.subcore_barrier()

    def load_idx(ci, b):
        off = ci * CC
        pltpu.sync_copy(src_hbm.at[pl.ds(off, CC)], srcv[b])
        pltpu.sync_copy(dst_hbm.at[pl.ds(off, CC)], dstv[b])
        pltpu.sync_copy(pT_hbm.at[pl.ds(off * H, CC * H)], pv[b])
        pltpu.async_copy(ys_hbm.at[srcv[b]], rows[b], sem[b])

    load_idx(base, 0)

    def do_chunk(i2, b, first):
        ci = base + i2 + b

        @pl.when(i2 + b + 1 < _CPT)
        def _():
            load_idx(ci + 1, 1 - b)

        pltpu.async_copy(invd_hbm.at[dstv[b]], ivd, sem[b])
        pltpu.make_async_copy(ys_hbm.at[srcv[b]], rows[b], sem[b]).wait()
        pltpu.make_async_copy(invd_hbm.at[dstv[b]], ivd, sem[b]).wait()
        @pl.when(jnp.logical_not(first))
        def _():
            pltpu.make_async_copy(msg[b], acc_sh.at[sdst[b]], ssem[b]).wait()

        rv = rows[b]
        pr = pv[b]
        ms = msg[b]

        @plsc.parallel_loop(0, CC // 4, 1, unroll=4)
        def _grp(g):
            iv = plsc.load_gather(ivd, [g * 4 + grow, gcol])
            wv = pr[pl.ds(g * 16, 16)] * iv
            for k in range(4):
                e = g * 4 + k
                ws = [_lane_take(wv, jnp.full((16,), 4 * k + h, _i32))
                      for h in range(H)]
                for j in range(D_OUT // 16):
                    m = ws[0] * rv[e, pl.ds(j * 16, 16)]
                    for h in range(1, H):
                        m = m + ws[h] * rv[e, pl.ds(h * C + j * 16, 16)]
                    ms[e, pl.ds(j * 16, 16)] = m

        sdst[b][pl.ds(0, 16)] = dstv[b][pl.ds(0, 16)]
        pltpu.async_copy(msg[b], acc_sh.at[sdst[b]], ssem[b], add=True)

    def chunk2(i, carry):
        for b in range(2):
            do_chunk(i * 2, b, first=(i == 0))
        return carry

    lax.fori_loop(0, _CPT // 2, chunk2, None)
    do_chunk(_CPT - 1, 0, first=False)
    pltpu.make_async_copy(msg[0], acc_sh.at[sdst[0]], ssem[0]).wait()
    pltpu.make_async_copy(msg[1], acc_sh.at[sdst[1]], ssem[1]).wait()
    plsc.subcore_barrier()
    _copy_out_rows(sid, acc_sh, out_hbm, cid * N)


def kernel(x, edge_index, edge_attr, W_src, att_src, att_dst, W_edge,
           att_edge, bias_gat, W_lin, b_lin, gamma, beta):
    src = edge_index[0]
    dst = edge_index[1]
    ys, meta = _prep_node(x, W_src, att_src, att_dst, W_lin)
    aeT = _prep_edge(edge_attr, W_edge, att_edge)
    z16 = jnp.zeros((N, 16), _f32)
    z128 = jnp.zeros((N, D_OUT), _f32)
    pT, accA = _pass_a(src, dst, aeT, meta.reshape(N * 2 * H), z16)
    invd16, aself = _mid(accA, meta)
    accC = _pass_c(src, dst, pT, invd16, ys, z128)
    return _post(x, accC, aself, ys, W_lin, bias_gat.reshape(1, HC),
                 b_lin.reshape(1, D_OUT), gamma.reshape(1, D_OUT),
                 beta.reshape(1, D_OUT))


# speedup vs baseline: 1.6561x; 1.6561x over previous
import functools

import jax
import jax.numpy as jnp
from jax import lax
from jax.experimental import pallas as pl
from jax.experimental.pallas import tpu as pltpu
from jax.experimental.pallas import tpu_sc as plsc

N = 10000
E = 320000
H = 4
C = 128
D_IN = 128
D_E = 16
D_OUT = 128
HC = H * C

_NC = 2
_NS = 16
_NW = _NC * _NS
CH = 128
NCHUNK = E // CH
_CPW = NCHUNK // _NW
_CREM = NCHUNK - _CPW * _NW
_RPT = 624
_RREM = N - _RPT * _NS
CC = 40
NCHUNK_C = E // CC
_CPT = NCHUNK_C // _NW

_f32 = jnp.float32
_i32 = jnp.int32


_BN = 1000


def _prep_node_body(x_ref, wsrc_ref, attsrc_ref, attdst_ref, wlin_ref,
                    ys_ref, meta_ref):
    xs = jnp.dot(x_ref[...], wsrc_ref[...], preferred_element_type=_f32)
    acols = []
    dcols = []
    for h in range(H):
        blk = xs[:, h * C:(h + 1) * C]
        acols.append((blk * attsrc_ref[h:h + 1, :]).sum(axis=1, keepdims=True))
        dcols.append((blk * attdst_ref[h:h + 1, :]).sum(axis=1, keepdims=True))
        ys_ref[:, h * C:(h + 1) * C] = jnp.dot(
            blk, wlin_ref[h * C:(h + 1) * C, :], preferred_element_type=_f32)
    meta_ref[...] = jnp.concatenate(acols + dcols, axis=1)


def _prep_node(x, W_src, att_src, att_dst, W_lin):
    return pl.pallas_call(
        _prep_node_body,
        grid=(N // _BN,),
        in_specs=[
            pl.BlockSpec((_BN, D_IN), lambda i: (i, 0)),
            pl.BlockSpec((D_IN, HC), lambda i: (0, 0)),
            pl.BlockSpec((H, C), lambda i: (0, 0)),
            pl.BlockSpec((H, C), lambda i: (0, 0)),
            pl.BlockSpec((HC, D_OUT), lambda i: (0, 0)),
        ],
        out_specs=[
            pl.BlockSpec((_BN, HC), lambda i: (i, 0)),
            pl.BlockSpec((_BN, 2 * H), lambda i: (i, 0)),
        ],
        out_shape=[
            jax.ShapeDtypeStruct((N, HC), _f32),
            jax.ShapeDtypeStruct((N, 2 * H), _f32),
        ],
    )(x, W_src, att_src, att_dst, W_lin)


_BE = 12800


def _prep_edge_body(ea_ref, wedge_ref, attedge_ref, out_ref):
    cols = []
    for h in range(H):
        cols.append((wedge_ref[:, h * C:(h + 1) * C]
                     * attedge_ref[h:h + 1, :]).sum(axis=1, keepdims=True))
    ae_fold = jnp.concatenate(cols, axis=1)
    out_ref[...] = lax.dot_general(
        ae_fold, ea_ref[...], (((0,), (1,)), ((), ())),
        preferred_element_type=_f32)


def _prep_edge(edge_attr, W_edge, att_edge):
    return pl.pallas_call(
        _prep_edge_body,
        grid=(E // _BE,),
        in_specs=[
            pl.BlockSpec((_BE, D_E), lambda i: (i, 0)),
            pl.BlockSpec((D_E, HC), lambda i: (0, 0)),
            pl.BlockSpec((H, C), lambda i: (0, 0)),
        ],
        out_specs=pl.BlockSpec((H, _BE), lambda i: (0, i)),
        out_shape=jax.ShapeDtypeStruct((H, E), _f32),
    )(edge_attr, W_edge, att_edge)


def _mid_body(acc_ref, meta_ref, invd_ref, aself_ref):
    acc = acc_ref[:N, :] + acc_ref[N:, :]
    denom_e = acc[:, 0:H]
    aesum = acc[:, H:2 * H]
    deg = acc[:, 2 * H:2 * H + 1]
    ae_self = aesum / jnp.maximum(deg, 1.0)
    a = meta_ref[:, 0:H] + meta_ref[:, H:2 * H] + ae_self
    a = jnp.where(a >= 0.0, a, 0.2 * a)
    p_self = jnp.exp(a)
    invd = 1.0 / (denom_e + p_self + 1e-16)
    invd_ref[...] = jnp.concatenate(
        [invd, jnp.zeros((N, 16 - H), _f32)], axis=1)
    aself_ref[...] = p_self * invd


def _mid(accA, meta):
    return pl.pallas_call(
        _mid_body,
        out_shape=[
            jax.ShapeDtypeStruct((N, 16), _f32),
            jax.ShapeDtypeStruct((N, H), _f32),
        ],
    )(accA, meta)


def _post_body(x_ref, a0_ref, a1_ref, aself_ref, ys_ref, wlin_ref,
               bg_ref, bl_ref, g_ref, b_ref, out_ref):
    c0 = jnp.dot(bg_ref[...], wlin_ref[...], preferred_element_type=_f32) \
        + bl_ref[...]
    st = a0_ref[...] + a1_ref[...]
    for h in range(H):
        st = st + aself_ref[:, h:h + 1] * ys_ref[:, h * C:(h + 1) * C]
    hh = x_ref[...] + st + c0
    mu = jnp.mean(hh, axis=1, keepdims=True)
    var = jnp.mean((hh - mu) ** 2, axis=1, keepdims=True)
    out_ref[...] = (hh - mu) * lax.rsqrt(var + 1e-5) * g_ref[...] + b_ref[...]


def _post(x, accC, aself, ys, W_lin, bias_gat, b_lin, gamma, beta):
    nb = N // _BN
    return pl.pallas_call(
        _post_body,
        grid=(nb,),
        in_specs=[
            pl.BlockSpec((_BN, D_IN), lambda i: (i, 0)),
            pl.BlockSpec((_BN, D_OUT), lambda i: (i, 0)),
            pl.BlockSpec((_BN, D_OUT), lambda i, _nb=nb: (i + _nb, 0)),
            pl.BlockSpec((_BN, H), lambda i: (i, 0)),
            pl.BlockSpec((_BN, HC), lambda i: (i, 0)),
            pl.BlockSpec((HC, D_OUT), lambda i: (0, 0)),
            pl.BlockSpec((1, HC), lambda i: (0, 0)),
            pl.BlockSpec((1, D_OUT), lambda i: (0, 0)),
            pl.BlockSpec((1, D_OUT), lambda i: (0, 0)),
            pl.BlockSpec((1, D_OUT), lambda i: (0, 0)),
        ],
        out_specs=pl.BlockSpec((_BN, D_OUT), lambda i: (i, 0)),
        out_shape=jax.ShapeDtypeStruct((N, D_OUT), _f32),
    )(x, accC, accC, aself, ys, W_lin, bias_gat, b_lin, gamma, beta)


_mesh = plsc.VectorSubcoreMesh(core_axis_name="c", subcore_axis_name="s")


def _tile_chunk_range(cid, sid, nchunk=NCHUNK):
    wid = sid * _NC + cid
    cpw = nchunk // _NW
    crem = nchunk - cpw * _NW
    start = wid * cpw + jnp.minimum(wid, crem)
    count = jnp.where(wid < crem, cpw + 1, cpw)
    return start, count


_TAKE_DN = lax.GatherDimensionNumbers(
    offset_dims=(), collapsed_slice_dims=(0,), start_index_map=(0,))


def _lane_take(x, idx):
    return lax.gather(
        x, idx[:, None], _TAKE_DN, slice_sizes=(1,),
        mode=lax.GatherScatterMode.PROMISE_IN_BOUNDS)


def _copy_out_rows(sid, sh_ref, hbm_ref, row0):
    pltpu.sync_copy(
        sh_ref.at[pl.ds(sid * _RPT, _RPT)],
        hbm_ref.at[pl.ds(row0 + sid * _RPT, _RPT)])

    @pl.when(sid == 0)
    def _():
        pltpu.sync_copy(
            sh_ref.at[pl.ds(_NS * _RPT, _RREM)],
            hbm_ref.at[pl.ds(row0 + _NS * _RPT, _RREM)])


CA = 80
NCHUNK_A = E // CA
_APT = NCHUNK_A // _NW


@functools.partial(
    pl.kernel,
    mesh=_mesh,
    compiler_params=pltpu.CompilerParams(
        needs_layout_passes=False, use_tc_tiling_on_sc=False),
    out_type=[
        jax.ShapeDtypeStruct((E * H,), _f32),
        jax.ShapeDtypeStruct((_NC * N, 16), _f32),
    ],
    scratch_types=(
        [pltpu.VMEM((N * 2 * H,), _f32)]
        + [pltpu.VMEM((CA,), _i32)] * 2
        + [pltpu.VMEM((CA,), _i32)] * 2
        + [pltpu.VMEM((H, CA), _f32)] * 2
        + [
            pltpu.VMEM((CA * H,), _f32),
            pltpu.VMEM((CA, 16), _f32),
            pltpu.VMEM_SHARED((N, 16), _f32),
            pltpu.SemaphoreType.DMA,
            pltpu.SemaphoreType.DMA,
        ]
    ),
)
def _pass_a(src_hbm, dst_hbm, aeT_hbm, meta_hbm, z16_hbm,
            pT_hbm, acc_hbm,
            meta_v, srcv0, srcv1, dstv0, dstv1, aev0, aev1,
            pst, upd, acc_sh, sem0, sem1):
    cid = lax.axis_index("c")
    sid = lax.axis_index("s")
    wid = sid * _NC + cid
    base = wid * _APT
    srcv = (srcv0, srcv1)
    dstv = (dstv0, dstv1)
    aev = (aev0, aev1)
    sem = (sem0, sem1)
    pltpu.sync_copy(meta_hbm, meta_v)
    zero16 = jnp.zeros((16,), _f32)
    for r in range(CA):
        upd[r, :] = zero16

    @pl.when(sid == 0)
    def _():
        pltpu.sync_copy(z16_hbm, acc_sh)

    plsc.subcore_barrier()
    iota16 = lax.iota(_i32, 16)
    ones16 = jnp.ones((16,), _f32)

    def load_in(ci, b):
        off = ci * CA
        pltpu.async_copy(src_hbm.at[pl.ds(off, CA)], srcv[b], sem[b])
        pltpu.async_copy(dst_hbm.at[pl.ds(off, CA)], dstv[b], sem[b])
        pltpu.async_copy(aeT_hbm.at[:, pl.ds(off, CA)], aev[b], sem[b])

    def wait_in(ci, b):
        off = ci * CA
        pltpu.make_async_copy(src_hbm.at[pl.ds(off, CA)], srcv[b],
                              sem[b]).wait()
        pltpu.make_async_copy(dst_hbm.at[pl.ds(off, CA)], dstv[b],
                              sem[b]).wait()
        pltpu.make_async_copy(aeT_hbm.at[:, pl.ds(off, CA)], aev[b],
                              sem[b]).wait()

    def compute(ci, b):
        off = ci * CA
        for g in range(CA // 16):
            s16 = srcv[b][pl.ds(g * 16, 16)]
            d16 = dstv[b][pl.ds(g * 16, 16)]
            rowi = iota16 + (g * 16)
            for h in range(H):
                a_s = plsc.load_gather(meta_v, [s16 * (2 * H) + h])
                a_d = plsc.load_gather(meta_v, [d16 * (2 * H) + (H + h)])
                ae = aev[b][h, pl.ds(g * 16, 16)]
                a = a_s + a_d + ae
                a = jnp.where(a >= 0.0, a, 0.2 * a)
                p = jnp.exp(a)
                plsc.store_scatter(pst, [rowi * H + h], p)
                plsc.store_scatter(
                    upd, [rowi, jnp.full((16,), h, _i32)], p)
                plsc.store_scatter(
                    upd, [rowi, jnp.full((16,), H + h, _i32)], ae)
            plsc.store_scatter(
                upd, [rowi, jnp.full((16,), 2 * H, _i32)], ones16)
        pltpu.sync_copy(pst, pT_hbm.at[pl.ds(off * H, CA * H)])
        pltpu.sync_copy(upd, acc_sh.at[dstv[b]], add=True)

    load_in(base, 0)

    def chunk2(i, carry):
        for b in range(2):
            ci = base + i * 2 + b
            load_in(ci + 1, 1 - b)
            wait_in(ci, b)
            compute(ci, b)
        return carry

    lax.fori_loop(0, (_APT - 1) // 2, chunk2, None)
    wait_in(base + _APT - 1, 0)
    compute(base + _APT - 1, 0)
    plsc.subcore_barrier()
    _copy_out_rows(sid, acc_sh, acc_hbm, cid * N)


@functools.partial(
    pl.kernel,
    mesh=_mesh,
    compiler_params=pltpu.CompilerParams(
        needs_layout_passes=False, use_tc_tiling_on_sc=False),
    out_type=jax.ShapeDtypeStruct((_NC * N, D_OUT), _f32),
    scratch_types=(
        [pltpu.VMEM((CC,), _i32)] * 2
        + [pltpu.VMEM((CC,), _i32)] * 2
        + [pltpu.VMEM((CC * H,), _f32)] * 2
        + [pltpu.VMEM((CC, 16), _f32)] * 2
        + [pltpu.VMEM((CC, HC), _f32)] * 2
        + [
            pltpu.VMEM((CC, D_OUT), _f32),
            pltpu.VMEM_SHARED((N, D_OUT), _f32),
            pltpu.SemaphoreType.DMA,
            pltpu.SemaphoreType.DMA,
        ]
    ),
)
def _pass_c(src_hbm, dst_hbm, pT_hbm, invd_hbm, ys_hbm, z128_hbm,
            out_hbm,
            srcv0, srcv1, dstv0, dstv1, pv0, pv1, ivd0, ivd1, rows0, rows1,
            msg, acc_sh, sem0, sem1):
    cid = lax.axis_index("c")
    sid = lax.axis_index("s")
    wid = sid * _NC + cid
    base = wid * _CPT
    srcv = (srcv0, srcv1)
    dstv = (dstv0, dstv1)
    pv = (pv0, pv1)
    ivd = (ivd0, ivd1)
    rows = (rows0, rows1)
    sem = (sem0, sem1)
    iota16 = lax.iota(_i32, 16)
    grow = iota16 // H
    gcol = iota16 - grow * H

    @pl.when(sid == 0)
    def _():
        pltpu.sync_copy(z128_hbm, acc_sh)

    plsc.subcore_barrier()

    def load_idx(ci, b):
        off = ci * CC
        pltpu.sync_copy(src_hbm.at[pl.ds(off, CC)], srcv[b])
        pltpu.sync_copy(dst_hbm.at[pl.ds(off, CC)], dstv[b])
        pltpu.sync_copy(pT_hbm.at[pl.ds(off * H, CC * H)], pv[b])
        pltpu.async_copy(invd_hbm.at[dstv[b]], ivd[b], sem[b])
        pltpu.async_copy(ys_hbm.at[srcv[b]], rows[b], sem[b])

    load_idx(base, 0)

    def chunk2(i, carry):
        for b in range(2):
            ci = base + i * 2 + b

            @pl.when(i * 2 + b + 1 < _CPT)
            def _():
                load_idx(ci + 1, 1 - b)

            pltpu.make_async_copy(invd_hbm.at[dstv[b]], ivd[b], sem[b]).wait()
            pltpu.make_async_copy(ys_hbm.at[srcv[b]], rows[b], sem[b]).wait()
            rv = rows[b]
            pr = pv[b]
            ir = ivd[b]

            @plsc.parallel_loop(0, CC // 4, 1, unroll=5)
            def _grp(g):
                iv = plsc.load_gather(ir, [g * 4 + grow, gcol])
                wv = pr[pl.ds(g * 16, 16)] * iv
                for k in range(4):
                    e = g * 4 + k
                    ws = [_lane_take(wv, jnp.full((16,), 4 * k + h, _i32))
                          for h in range(H)]
                    for j in range(D_OUT // 16):
                        m = ws[0] * rv[e, pl.ds(j * 16, 16)]
                        for h in range(1, H):
                            m = m + ws[h] * rv[e, pl.ds(h * C + j * 16, 16)]
                        msg[e, pl.ds(j * 16, 16)] = m

            pltpu.sync_copy(msg, acc_sh.at[dstv[b]], add=True)
        return carry

    lax.fori_loop(0, _CPT // 2, chunk2, None)
    plsc.subcore_barrier()
    _copy_out_rows(sid, acc_sh, out_hbm, cid * N)


def kernel(x, edge_index, edge_attr, W_src, att_src, att_dst, W_edge,
           att_edge, bias_gat, W_lin, b_lin, gamma, beta):
    src = edge_index[0]
    dst = edge_index[1]
    ys, meta = _prep_node(x, W_src, att_src, att_dst, W_lin)
    aeT = _prep_edge(edge_attr, W_edge, att_edge)
    z16 = jnp.zeros((N, 16), _f32)
    z128 = jnp.zeros((N, D_OUT), _f32)
    pT, accA = _pass_a(src, dst, aeT, meta.reshape(N * 2 * H), z16)
    invd16, aself = _mid(accA, meta)
    accC = _pass_c(src, dst, pT, invd16, ys, z128)
    return _post(x, accC, aself, ys, W_lin, bias_gat.reshape(1, HC),
                 b_lin.reshape(1, D_OUT), gamma.reshape(1, D_OUT),
                 beta.reshape(1, D_OUT))


# speedup vs baseline: 1.6566x; 1.0003x over previous
import functools

import jax
import jax.numpy as jnp
from jax import lax
from jax.experimental import pallas as pl
from jax.experimental.pallas import tpu as pltpu
from jax.experimental.pallas import tpu_sc as plsc

N = 10000
E = 320000
H = 4
C = 128
D_IN = 128
D_E = 16
D_OUT = 128
HC = H * C

_NC = 2
_NS = 16
_NW = _NC * _NS
_RPT = 624
_RREM = N - _RPT * _NS
CC = 40
_CPT = E // CC // _NW

_f32 = jnp.float32
_i32 = jnp.int32


_BN = 1000


def _prep_node_body(x_ref, wsrc_ref, attsrc_ref, attdst_ref, wlin_ref,
                    ys_ref, meta_ref):
    xs = jnp.dot(x_ref[...], wsrc_ref[...], preferred_element_type=_f32)
    acols = []
    dcols = []
    for h in range(H):
        blk = xs[:, h * C:(h + 1) * C]
        acols.append((blk * attsrc_ref[h:h + 1, :]).sum(axis=1, keepdims=True))
        dcols.append((blk * attdst_ref[h:h + 1, :]).sum(axis=1, keepdims=True))
        ys_ref[:, h * C:(h + 1) * C] = jnp.dot(
            blk, wlin_ref[h * C:(h + 1) * C, :], preferred_element_type=_f32)
    meta_ref[...] = jnp.concatenate(acols + dcols, axis=1)


def _prep_node(x, W_src, att_src, att_dst, W_lin):
    return pl.pallas_call(
        _prep_node_body,
        grid=(N // _BN,),
        in_specs=[
            pl.BlockSpec((_BN, D_IN), lambda i: (i, 0)),
            pl.BlockSpec((D_IN, HC), lambda i: (0, 0)),
            pl.BlockSpec((H, C), lambda i: (0, 0)),
            pl.BlockSpec((H, C), lambda i: (0, 0)),
            pl.BlockSpec((HC, D_OUT), lambda i: (0, 0)),
        ],
        out_specs=[
            pl.BlockSpec((_BN, HC), lambda i: (i, 0)),
            pl.BlockSpec((_BN, 2 * H), lambda i: (i, 0)),
        ],
        out_shape=[
            jax.ShapeDtypeStruct((N, HC), _f32),
            jax.ShapeDtypeStruct((N, 2 * H), _f32),
        ],
    )(x, W_src, att_src, att_dst, W_lin)


_BE = 12800


def _prep_edge_body(ea_ref, wedge_ref, attedge_ref, out_ref):
    cols = []
    for h in range(H):
        cols.append((wedge_ref[:, h * C:(h + 1) * C]
                     * attedge_ref[h:h + 1, :]).sum(axis=1, keepdims=True))
    ae_fold = jnp.concatenate(cols, axis=1)
    out_ref[...] = lax.dot_general(
        ae_fold, ea_ref[...], (((0,), (1,)), ((), ())),
        preferred_element_type=_f32)


def _prep_edge(edge_attr, W_edge, att_edge):
    return pl.pallas_call(
        _prep_edge_body,
        grid=(E // _BE,),
        in_specs=[
            pl.BlockSpec((_BE, D_E), lambda i: (i, 0)),
            pl.BlockSpec((D_E, HC), lambda i: (0, 0)),
            pl.BlockSpec((H, C), lambda i: (0, 0)),
        ],
        out_specs=pl.BlockSpec((H, _BE), lambda i: (0, i)),
        out_shape=jax.ShapeDtypeStruct((H, E), _f32),
    )(edge_attr, W_edge, att_edge)


def _mid_body(acc_ref, meta_ref, invd_ref, aself_ref):
    acc = acc_ref[:N, :] + acc_ref[N:, :]
    denom_e = acc[:, 0:H]
    aesum = acc[:, H:2 * H]
    deg = acc[:, 2 * H:2 * H + 1]
    ae_self = aesum / jnp.maximum(deg, 1.0)
    a = meta_ref[:, 0:H] + meta_ref[:, H:2 * H] + ae_self
    a = jnp.where(a >= 0.0, a, 0.2 * a)
    p_self = jnp.exp(a)
    invd = 1.0 / (denom_e + p_self + 1e-16)
    invd_ref[...] = jnp.concatenate(
        [invd, jnp.zeros((N, 16 - H), _f32)], axis=1)
    aself_ref[...] = p_self * invd


def _mid(accA, meta):
    return pl.pallas_call(
        _mid_body,
        out_shape=[
            jax.ShapeDtypeStruct((N, 16), _f32),
            jax.ShapeDtypeStruct((N, H), _f32),
        ],
    )(accA, meta)


def _post_body(x_ref, a0_ref, a1_ref, aself_ref, ys_ref, wlin_ref,
               bg_ref, bl_ref, g_ref, b_ref, out_ref):
    c0 = jnp.dot(bg_ref[...], wlin_ref[...], preferred_element_type=_f32) \
        + bl_ref[...]
    st = a0_ref[...] + a1_ref[...]
    for h in range(H):
        st = st + aself_ref[:, h:h + 1] * ys_ref[:, h * C:(h + 1) * C]
    hh = x_ref[...] + st + c0
    mu = jnp.mean(hh, axis=1, keepdims=True)
    var = jnp.mean((hh - mu) ** 2, axis=1, keepdims=True)
    out_ref[...] = (hh - mu) * lax.rsqrt(var + 1e-5) * g_ref[...] + b_ref[...]


def _post(x, accC, aself, ys, W_lin, bias_gat, b_lin, gamma, beta):
    nb = N // _BN
    return pl.pallas_call(
        _post_body,
        grid=(nb,),
        in_specs=[
            pl.BlockSpec((_BN, D_IN), lambda i: (i, 0)),
            pl.BlockSpec((_BN, D_OUT), lambda i: (i, 0)),
            pl.BlockSpec((_BN, D_OUT), lambda i, _nb=nb: (i + _nb, 0)),
            pl.BlockSpec((_BN, H), lambda i: (i, 0)),
            pl.BlockSpec((_BN, HC), lambda i: (i, 0)),
            pl.BlockSpec((HC, D_OUT), lambda i: (0, 0)),
            pl.BlockSpec((1, HC), lambda i: (0, 0)),
            pl.BlockSpec((1, D_OUT), lambda i: (0, 0)),
            pl.BlockSpec((1, D_OUT), lambda i: (0, 0)),
            pl.BlockSpec((1, D_OUT), lambda i: (0, 0)),
        ],
        out_specs=pl.BlockSpec((_BN, D_OUT), lambda i: (i, 0)),
        out_shape=jax.ShapeDtypeStruct((N, D_OUT), _f32),
    )(x, accC, accC, aself, ys, W_lin, bias_gat, b_lin, gamma, beta)


_mesh = plsc.VectorSubcoreMesh(core_axis_name="c", subcore_axis_name="s")


_TAKE_DN = lax.GatherDimensionNumbers(
    offset_dims=(), collapsed_slice_dims=(0,), start_index_map=(0,))


def _lane_take(x, idx):
    return lax.gather(
        x, idx[:, None], _TAKE_DN, slice_sizes=(1,),
        mode=lax.GatherScatterMode.PROMISE_IN_BOUNDS)


def _copy_out_rows(sid, sh_ref, hbm_ref, row0):
    pltpu.sync_copy(
        sh_ref.at[pl.ds(sid * _RPT, _RPT)],
        hbm_ref.at[pl.ds(row0 + sid * _RPT, _RPT)])

    @pl.when(sid == 0)
    def _():
        pltpu.sync_copy(
            sh_ref.at[pl.ds(_NS * _RPT, _RREM)],
            hbm_ref.at[pl.ds(row0 + _NS * _RPT, _RREM)])


CA = 80
NCHUNK_A = E // CA
_APT = NCHUNK_A // _NW


@functools.partial(
    pl.kernel,
    mesh=_mesh,
    compiler_params=pltpu.CompilerParams(
        needs_layout_passes=False, use_tc_tiling_on_sc=False),
    out_type=[
        jax.ShapeDtypeStruct((E * H,), _f32),
        jax.ShapeDtypeStruct((_NC * N, 16), _f32),
    ],
    scratch_types=(
        [pltpu.VMEM((N * 2 * H,), _f32)]
        + [pltpu.VMEM((CA,), _i32)] * 2
        + [pltpu.VMEM((CA,), _i32)] * 2
        + [pltpu.VMEM((H, CA), _f32)] * 2
        + [
            pltpu.VMEM((CA * H,), _f32),
            pltpu.VMEM((CA, 16), _f32),
            pltpu.VMEM_SHARED((N, 16), _f32),
            pltpu.SemaphoreType.DMA,
            pltpu.SemaphoreType.DMA,
        ]
    ),
)
def _pass_a(src_hbm, dst_hbm, aeT_hbm, meta_hbm, z16_hbm,
            pT_hbm, acc_hbm,
            meta_v, srcv0, srcv1, dstv0, dstv1, aev0, aev1,
            pst, upd, acc_sh, sem0, sem1):
    cid = lax.axis_index("c")
    sid = lax.axis_index("s")
    wid = sid * _NC + cid
    base = wid * _APT
    srcv = (srcv0, srcv1)
    dstv = (dstv0, dstv1)
    aev = (aev0, aev1)
    sem = (sem0, sem1)
    pltpu.sync_copy(meta_hbm, meta_v)
    zero16 = jnp.zeros((16,), _f32)
    for r in range(CA):
        upd[r, :] = zero16

    @pl.when(sid == 0)
    def _():
        pltpu.sync_copy(z16_hbm, acc_sh)

    plsc.subcore_barrier()
    iota16 = lax.iota(_i32, 16)
    ones16 = jnp.ones((16,), _f32)

    def load_in(ci, b):
        off = ci * CA
        pltpu.async_copy(src_hbm.at[pl.ds(off, CA)], srcv[b], sem[b])
        pltpu.async_copy(dst_hbm.at[pl.ds(off, CA)], dstv[b], sem[b])
        pltpu.async_copy(aeT_hbm.at[:, pl.ds(off, CA)], aev[b], sem[b])

    def wait_in(ci, b):
        off = ci * CA
        pltpu.make_async_copy(src_hbm.at[pl.ds(off, CA)], srcv[b],
                              sem[b]).wait()
        pltpu.make_async_copy(dst_hbm.at[pl.ds(off, CA)], dstv[b],
                              sem[b]).wait()
        pltpu.make_async_copy(aeT_hbm.at[:, pl.ds(off, CA)], aev[b],
                              sem[b]).wait()

    def compute(ci, b):
        off = ci * CA
        for g in range(CA // 16):
            s16 = srcv[b][pl.ds(g * 16, 16)]
            d16 = dstv[b][pl.ds(g * 16, 16)]
            rowi = iota16 + (g * 16)
            for h in range(H):
                a_s = plsc.load_gather(meta_v, [s16 * (2 * H) + h])
                a_d = plsc.load_gather(meta_v, [d16 * (2 * H) + (H + h)])
                ae = aev[b][h, pl.ds(g * 16, 16)]
                a = a_s + a_d + ae
                a = jnp.where(a >= 0.0, a, 0.2 * a)
                p = jnp.exp(a)
                plsc.store_scatter(pst, [rowi * H + h], p)
                plsc.store_scatter(
                    upd, [rowi, jnp.full((16,), h, _i32)], p)
                plsc.store_scatter(
                    upd, [rowi, jnp.full((16,), H + h, _i32)], ae)
            plsc.store_scatter(
                upd, [rowi, jnp.full((16,), 2 * H, _i32)], ones16)
        pltpu.sync_copy(pst, pT_hbm.at[pl.ds(off * H, CA * H)])
        pltpu.sync_copy(upd, acc_sh.at[dstv[b]], add=True)

    load_in(base, 0)

    def chunk2(i, carry):
        for b in range(2):
            ci = base + i * 2 + b
            load_in(ci + 1, 1 - b)
            wait_in(ci, b)
            compute(ci, b)
        return carry

    lax.fori_loop(0, (_APT - 1) // 2, chunk2, None)
    wait_in(base + _APT - 1, 0)
    compute(base + _APT - 1, 0)
    plsc.subcore_barrier()
    _copy_out_rows(sid, acc_sh, acc_hbm, cid * N)


@functools.partial(
    pl.kernel,
    mesh=_mesh,
    compiler_params=pltpu.CompilerParams(
        needs_layout_passes=False, use_tc_tiling_on_sc=False),
    out_type=jax.ShapeDtypeStruct((_NC * N, D_OUT), _f32),
    scratch_types=(
        [pltpu.VMEM((CC,), _i32)] * 2
        + [pltpu.VMEM((CC,), _i32)] * 2
        + [pltpu.VMEM((CC * H,), _f32)] * 2
        + [pltpu.VMEM((CC, 16), _f32)] * 2
        + [pltpu.VMEM((CC, HC), _f32)] * 2
        + [
            pltpu.VMEM((CC, D_OUT), _f32),
            pltpu.VMEM_SHARED((N, D_OUT), _f32),
            pltpu.SemaphoreType.DMA,
            pltpu.SemaphoreType.DMA,
        ]
    ),
)
def _pass_c(src_hbm, dst_hbm, pT_hbm, invd_hbm, ys_hbm, z128_hbm,
            out_hbm,
            srcv0, srcv1, dstv0, dstv1, pv0, pv1, ivd0, ivd1, rows0, rows1,
            msg, acc_sh, sem0, sem1):
    cid = lax.axis_index("c")
    sid = lax.axis_index("s")
    wid = sid * _NC + cid
    base = wid * _CPT
    srcv = (srcv0, srcv1)
    dstv = (dstv0, dstv1)
    pv = (pv0, pv1)
    ivd = (ivd0, ivd1)
    rows = (rows0, rows1)
    sem = (sem0, sem1)
    iota16 = lax.iota(_i32, 16)
    grow = iota16 // H
    gcol = iota16 - grow * H

    @pl.when(sid == 0)
    def _():
        pltpu.sync_copy(z128_hbm, acc_sh)

    plsc.subcore_barrier()

    def load_idx(ci, b):
        off = ci * CC
        pltpu.sync_copy(src_hbm.at[pl.ds(off, CC)], srcv[b])
        pltpu.sync_copy(dst_hbm.at[pl.ds(off, CC)], dstv[b])
        pltpu.sync_copy(pT_hbm.at[pl.ds(off * H, CC * H)], pv[b])
        pltpu.async_copy(invd_hbm.at[dstv[b]], ivd[b], sem[b])
        pltpu.async_copy(ys_hbm.at[srcv[b]], rows[b], sem[b])

    load_idx(base, 0)

    def chunk2(i, carry):
        for b in range(2):
            ci = base + i * 2 + b

            @pl.when(i * 2 + b + 1 < _CPT)
            def _():
                load_idx(ci + 1, 1 - b)

            pltpu.make_async_copy(invd_hbm.at[dstv[b]], ivd[b], sem[b]).wait()
            pltpu.make_async_copy(ys_hbm.at[srcv[b]], rows[b], sem[b]).wait()
            rv = rows[b]
            pr = pv[b]
            ir = ivd[b]

            @plsc.parallel_loop(0, CC // 4, 1, unroll=5)
            def _grp(g):
                iv = plsc.load_gather(ir, [g * 4 + grow, gcol])
                wv = pr[pl.ds(g * 16, 16)] * iv
                for k in range(4):
                    e = g * 4 + k
                    ws = [_lane_take(wv, jnp.full((16,), 4 * k + h, _i32))
                          for h in range(H)]
                    for j in range(D_OUT // 16):
                        m = ws[0] * rv[e, pl.ds(j * 16, 16)]
                        for h in range(1, H):
                            m = m + ws[h] * rv[e, pl.ds(h * C + j * 16, 16)]
                        msg[e, pl.ds(j * 16, 16)] = m

            pltpu.sync_copy(msg, acc_sh.at[dstv[b]], add=True)
        return carry

    lax.fori_loop(0, _CPT // 2, chunk2, None)
    plsc.subcore_barrier()
    _copy_out_rows(sid, acc_sh, out_hbm, cid * N)


def kernel(x, edge_index, edge_attr, W_src, att_src, att_dst, W_edge,
           att_edge, bias_gat, W_lin, b_lin, gamma, beta):
    src = edge_index[0]
    dst = edge_index[1]
    ys, meta = _prep_node(x, W_src, att_src, att_dst, W_lin)
    aeT = _prep_edge(edge_attr, W_edge, att_edge)
    z16 = jnp.zeros((N, 16), _f32)
    z128 = jnp.zeros((N, D_OUT), _f32)
    pT, accA = _pass_a(src, dst, aeT, meta.reshape(N * 2 * H), z16)
    invd16, aself = _mid(accA, meta)
    accC = _pass_c(src, dst, pT, invd16, ys, z128)
    return _post(x, accC, aself, ys, W_lin, bias_gat.reshape(1, HC),
                 b_lin.reshape(1, D_OUT), gamma.reshape(1, D_OUT),
                 beta.reshape(1, D_OUT))
